# baseline (speedup 1.0000x reference)
import jax
import jax.numpy as jnp
from jax import lax
from jax.experimental import pallas as pl
from jax.experimental.pallas import tpu as pltpu
from jax.experimental.pallas import tpu_sc as plsc

_B, _N, _K = 2, 10000, 16
_NPAD = 10240
_NROW = _B * _NPAD
_NE = _NROW * _K
_NW = 32
_EPW = _NE // _NW
_CH = 128
_NCH = _EPW // _CH
_RN = 256
_NCB = _NROW // _RN
_R = 1024
_NBLK = _NROW // _R
_EPS = 1e-5
_CNT2D = float(_B * _N * _K)
_CNT1D = float(_B * _N)


def _lrelu(v):
    return jnp.where(v >= 0, v, 0.2 * v)


def _row_mask(i, rows_per_block, v):
    rows = i * rows_per_block + lax.broadcasted_iota(
        jnp.int32, (rows_per_block, 1), 0)
    return jnp.where(rows % _NPAD < _N, v, 0.0)


def _dot(a, w):
    return lax.dot_general(a, w, (((1,), (1,)), ((), ())),
                           preferred_element_type=jnp.float32)


def _scale_shift(p_ref, g_ref, b_ref, cnt):
    p = p_ref[...]
    s = jnp.sum(p[:, 0, :], 0)
    s2 = jnp.sum(p[:, 1, :], 0)
    mean = s / cnt
    var = s2 / cnt - mean * mean
    scale = g_ref[...] * (1.0 / jnp.sqrt(var + _EPS))
    shift = b_ref[...] - mean * scale
    return scale, shift


def _sc_gather_rows(tab, idxkm, half):
    C = tab.shape[1]
    NH = _NROW // 2
    NQ = _NROW // 4
    NCHH = NQ // _CH
    mesh = plsc.VectorSubcoreMesh(core_axis_name="c", subcore_axis_name="s")

    def kbody(tab_hbm, idx_hbm, out_hbm,
              Iall, R0, R1, R2, R3,
              gs0, gs1, gs2, gs3, ds0, ds1, ds2, ds3):
        cid = lax.axis_index("c")
        sid = lax.axis_index("s")
        wid = sid * 2 + cid
        k0 = wid // 2
        rbase = (wid % 2) * NQ

        row0 = (k0 * _NROW + half * NH + rbase) // _CH
        pltpu.sync_copy(idx_hbm.at[pl.ds(row0, NCHH)], Iall)

        def start(c, Rw, gs):
            pltpu.async_copy(tab_hbm.at[Iall.at[c]], Rw, gs)

        start(0, R0, gs0)
        start(1, R1, gs1)

        def dst(c):
            return out_hbm.at[k0, pl.ds(rbase + c * _CH, _CH), pl.ds(0, C)]

        def step(c, Rw, gs, ds, Rn, gsn, dsn):
            pltpu.make_async_copy(tab_hbm.at[Iall.at[c]], Rw, gs).wait()
            pltpu.async_copy(Rw, dst(c), ds)

            @pl.when(c + 2 < NCHH)
            def _():
                @pl.when(c >= 2)
                def _():
                    pltpu.make_async_copy(Rn, dst(c), dsn).wait()
                start(c + 2, Rn, gsn)

        @pl.loop(0, NCHH, step=4)
        def lp(c):
            step(c + 0, R0, gs0, ds0, R2, gs2, ds2)
            step(c + 1, R1, gs1, ds1, R3, gs3, ds3)
            step(c + 2, R2, gs2, ds2, R0, gs0, ds0)
            step(c + 3, R3, gs3, ds3, R1, gs1, ds1)

        pltpu.make_async_copy(R0, dst(0), ds0).wait()
        pltpu.make_async_copy(R1, dst(0), ds1).wait()
        pltpu.make_async_copy(R2, dst(0), ds2).wait()
        pltpu.make_async_copy(R3, dst(0), ds3).wait()

    return pl.kernel(
        kbody,
        out_type=jax.ShapeDtypeStruct((_K, NH, 128), jnp.float32),
        mesh=mesh,
        compiler_params=pltpu.CompilerParams(use_tc_tiling_on_sc=False),
        scratch_types=(
            [pltpu.VMEM((NCHH, _CH), jnp.int32)]
            + [pltpu.VMEM((_CH, C), jnp.float32)] * 4
            + [pltpu.SemaphoreType.DMA] * 8
        ),
    )(tab, idxkm)


def _tc_conv_call(gath3, tab, W, half):
    O = W.shape[0]
    C = tab.shape[1]
    NH = _NROW // 2
    NCBH = NH // _RN
    boff = half * NCBH

    def body(g_ref, x_ref, w_ref, m_ref, p_ref):
        own = x_ref[...]
        if C < 128:
            own = jnp.concatenate(
                [own, jnp.zeros((_RN, 128 - C), jnp.float32)], axis=1)
        w = w_ref[...]
        lane = lax.broadcasted_iota(jnp.int32, (_K * _RN, 128), 1)
        d = (g_ref[...] - own[None, :, :]).reshape(_K * _RN, 128)
        d = jnp.where(lane < C, d, 0.0)
        p = _dot(d, w)
        m = p[0:_RN]
        for k in range(1, _K):
            m = jnp.maximum(m, p[k * _RN:(k + 1) * _RN])
        m_ref[...] = m
        p_ref[...] = jnp.stack([jnp.sum(p, 0), jnp.sum(p * p, 0)])[None]

    return pl.pallas_call(
        body,
        grid=(NCBH,),
        in_specs=[pl.BlockSpec((_K, _RN, 128), lambda i: (0, i, 0)),
                  pl.BlockSpec((_RN, C), lambda i: (i + boff, 0)),
                  pl.BlockSpec((O, 128), lambda i: (0, 0))],
        out_specs=[pl.BlockSpec((_RN, O), lambda i: (i, 0)),
                   pl.BlockSpec((1, 2, O), lambda i: (i, 0, 0))],
        out_shape=[jax.ShapeDtypeStruct((NH, O), jnp.float32),
                   jax.ShapeDtypeStruct((NCBH, 2, O), jnp.float32)],
    )(gath3, tab, W)


def _tc_apply_call(M, P, g, b):
    O = M.shape[1]

    def body(m_ref, p_ref, g_ref, b_ref, o_ref):
        i = pl.program_id(0)
        scale, shift = _scale_shift(p_ref, g_ref, b_ref, _CNT2D)
        o_ref[...] = _row_mask(i, _R, _lrelu(m_ref[...] * scale[None, :]
                                             + shift[None, :]))

    return pl.pallas_call(
        body,
        grid=(_NBLK,),
        in_specs=[pl.BlockSpec((_R, O), lambda i: (i, 0)),
                  pl.BlockSpec((_NCB, 2, O), lambda i: (0, 0, 0)),
                  pl.BlockSpec((O,), lambda i: (0,)),
                  pl.BlockSpec((O,), lambda i: (0,))],
        out_specs=pl.BlockSpec((_R, O), lambda i: (i, 0)),
        out_shape=jax.ShapeDtypeStruct((_NROW, O), jnp.float32),
    )(M, P, g, b)


def _tc_apply4_call(M, P, g, b, x1, x2, x3, W5a, W5b, W5c, W5d):
    O = M.shape[1]
    On = W5a.shape[0]

    def body(m_ref, p_ref, g_ref, b_ref, x1_ref, x2_ref, x3_ref,
             w5a_ref, w5b_ref, w5c_ref, w5d_ref, y_ref, pout_ref):
        i = pl.program_id(0)
        scale, shift = _scale_shift(p_ref, g_ref, b_ref, _CNT2D)
        x4v = _row_mask(i, _R, _lrelu(m_ref[...] * scale[None, :]
                                      + shift[None, :]))
        y = (_dot(x1_ref[...], w5a_ref[...])
             + _dot(x2_ref[...], w5b_ref[...])
             + _dot(x3_ref[...], w5c_ref[...])
             + _dot(x4v, w5d_ref[...]))
        y_ref[...] = y
        pout_ref[...] = jnp.stack([jnp.sum(y, 0), jnp.sum(y * y, 0)])[None]

    return pl.pallas_call(
        body,
        grid=(_NBLK,),
        in_specs=[pl.BlockSpec((_R, O), lambda i: (i, 0)),
                  pl.BlockSpec((_NCB, 2, O), lambda i: (0, 0, 0)),
                  pl.BlockSpec((O,), lambda i: (0,)),
                  pl.BlockSpec((O,), lambda i: (0,)),
                  pl.BlockSpec((_R, 64), lambda i: (i, 0)),
                  pl.BlockSpec((_R, 64), lambda i: (i, 0)),
                  pl.BlockSpec((_R, 128), lambda i: (i, 0)),
                  pl.BlockSpec((On, 64), lambda i: (0, 0)),
                  pl.BlockSpec((On, 64), lambda i: (0, 0)),
                  pl.BlockSpec((On, 128), lambda i: (0, 0)),
                  pl.BlockSpec((On, 256), lambda i: (0, 0))],
        out_specs=[pl.BlockSpec((_R, On), lambda i: (i, 0)),
                   pl.BlockSpec((1, 2, On), lambda i: (i, 0, 0))],
        out_shape=[jax.ShapeDtypeStruct((_NROW, On), jnp.float32),
                   jax.ShapeDtypeStruct((_NBLK, 2, On), jnp.float32)],
    )(M, P, g, b, x1, x2, x3, W5a, W5b, W5c, W5d)


def _apply_mlp_call(Hpre, P, g, b, Wn):
    O = Hpre.shape[1]
    On = Wn.shape[0]
    npart = P.shape[0]

    def body(h_ref, p_ref, g_ref, b_ref, w_ref, y_ref, pout_ref):
        i = pl.program_id(0)
        scale, shift = _scale_shift(p_ref, g_ref, b_ref, _CNT1D)
        hv = _row_mask(i, _R, _lrelu(h_ref[...] * scale[None, :]
                                     + shift[None, :]))
        y = _dot(hv, w_ref[...])
        y_ref[...] = y
        pout_ref[...] = jnp.stack([jnp.sum(y, 0), jnp.sum(y * y, 0)])[None]

    return pl.pallas_call(
        body,
        grid=(_NBLK,),
        in_specs=[pl.BlockSpec((_R, O), lambda i: (i, 0)),
                  pl.BlockSpec((npart, 2, O), lambda i: (0, 0, 0)),
                  pl.BlockSpec((O,), lambda i: (0,)),
                  pl.BlockSpec((O,), lambda i: (0,)),
                  pl.BlockSpec((On, O), lambda i: (0, 0))],
        out_specs=[pl.BlockSpec((_R, On), lambda i: (i, 0)),
                   pl.BlockSpec((1, 2, On), lambda i: (i, 0, 0))],
        out_shape=[jax.ShapeDtypeStruct((_NROW, On), jnp.float32),
                   jax.ShapeDtypeStruct((_NBLK, 2, On), jnp.float32)],
    )(Hpre, P, g, b, Wn)


def _final_call(Hpre, P, g, b, W7):
    O = Hpre.shape[1]

    def body(h_ref, p_ref, g_ref, b_ref, w_ref, o_ref):
        scale, shift = _scale_shift(p_ref, g_ref, b_ref, _CNT1D)
        hv = _lrelu(h_ref[...] * scale[None, :] + shift[None, :])
        o_ref[...] = _lrelu(_dot(hv, w_ref[...]))

    return pl.pallas_call(
        body,
        grid=(_NBLK,),
        in_specs=[pl.BlockSpec((_R, O), lambda i: (i, 0)),
                  pl.BlockSpec((_NBLK, 2, O), lambda i: (0, 0, 0)),
                  pl.BlockSpec((O,), lambda i: (0,)),
                  pl.BlockSpec((O,), lambda i: (0,)),
                  pl.BlockSpec((1, O), lambda i: (0, 0))],
        out_specs=pl.BlockSpec((_R, 1), lambda i: (i, 0)),
        out_shape=jax.ShapeDtypeStruct((_NROW, 1), jnp.float32),
    )(Hpre, P, g, b, W7)


@jax.jit
def kernel(x, idx, W1, W2, W3, W4, W5, W6, W7,
           g1, b1, g2, b2, g3, b3, g4, b4, g5, b5, g6, b6):
    xT = jnp.transpose(x, (0, 2, 1))
    xT = jnp.pad(xT, ((0, 0), (0, _NPAD - _N), (0, 13)))
    T1 = xT.reshape(_NROW, 16)

    W1p = jnp.pad(W1, ((0, 0), (0, 125)))
    W2p = jnp.pad(W2, ((0, 0), (0, 64)))
    W3p = jnp.pad(W3, ((0, 0), (0, 64)))
    W5a = W5[:, 0:64]
    W5b = W5[:, 64:128]
    W5c = W5[:, 128:256]
    W5d = W5[:, 256:512]

    idxp = jnp.pad(idx, ((0, 0), (0, _NPAD - _N), (0, 0)), constant_values=_N)
    idxg = idxp + (jnp.arange(_B, dtype=jnp.int32) * _NPAD)[:, None, None]
    idxkm = jnp.transpose(idxg.reshape(_NROW, _K), (1, 0)).reshape(-1, _CH)

    def edge_layer(T, W, g, b):
        GA = _sc_gather_rows(T, idxkm, 0)
        GB = _sc_gather_rows(T, idxkm, 1)
        MA, PA = _tc_conv_call(GA, T, W, 0)
        MB, PB = _tc_conv_call(GB, T, W, 1)
        return (jnp.concatenate([MA, MB], axis=0),
                jnp.concatenate([PA, PB], axis=0))

    M1, P1 = edge_layer(T1, W1p, g1, b1)
    T2 = _tc_apply_call(M1, P1, g1, b1)
    M2, P2 = edge_layer(T2, W2p, g2, b2)
    T3 = _tc_apply_call(M2, P2, g2, b2)
    M3, P3 = edge_layer(T3, W3p, g3, b3)
    T4 = _tc_apply_call(M3, P3, g3, b3)
    M4, P4 = edge_layer(T4, W4, g4, b4)
    h5pre, P5 = _tc_apply4_call(M4, P4, g4, b4, T2, T3, T4,
                                W5a, W5b, W5c, W5d)
    h6pre, P6 = _apply_mlp_call(h5pre, P5, g5, b5, W6)
    out = _final_call(h6pre, P6, g6, b6, W7)
    return out.reshape(_B, _NPAD, 1)[:, :_N, :]


# speedup vs baseline: 1.2426x; 1.2426x over previous
import jax
import jax.numpy as jnp
from jax import lax
from jax.experimental import pallas as pl
from jax.experimental.pallas import tpu as pltpu
from jax.experimental.pallas import tpu_sc as plsc

_B, _N, _K = 2, 10000, 16
_NPAD = 10240
_NROW = _B * _NPAD
_NE = _NROW * _K
_NW = 32
_EPW = _NE // _NW
_CH = 128
_NCH = _EPW // _CH
_RN = 256
_NCB = _NROW // _RN
_R = 1024
_NBLK = _NROW // _R
_EPS = 1e-5
_CNT2D = float(_B * _N * _K)
_CNT1D = float(_B * _N)


def _lrelu(v):
    return jnp.where(v >= 0, v, 0.2 * v)


def _row_mask(i, rows_per_block, v):
    rows = i * rows_per_block + lax.broadcasted_iota(
        jnp.int32, (rows_per_block, 1), 0)
    return jnp.where(rows % _NPAD < _N, v, 0.0)


def _dot(a, w):
    return lax.dot_general(a, w, (((1,), (1,)), ((), ())),
                           preferred_element_type=jnp.float32)


def _scale_shift(p_ref, g_ref, b_ref, cnt):
    p = p_ref[...]
    s = jnp.sum(p[:, 0, :], 0)
    s2 = jnp.sum(p[:, 1, :], 0)
    mean = s / cnt
    var = s2 / cnt - mean * mean
    scale = g_ref[...] * (1.0 / jnp.sqrt(var + _EPS))
    shift = b_ref[...] - mean * scale
    return scale, shift


def _sc_gather_rows(tab, idxkm):
    C = tab.shape[1]
    NCHH = _NCH
    mesh = plsc.VectorSubcoreMesh(core_axis_name="c", subcore_axis_name="s")

    def kbody(tab_hbm, idx_hbm, out_hbm,
              Iall, R0, R1, R2, R3,
              gs0, gs1, gs2, gs3, ds0, ds1, ds2, ds3):
        cid = lax.axis_index("c")
        sid = lax.axis_index("s")
        wid = sid * 2 + cid
        k0 = wid // 2
        rbase = (wid % 2) * (_NROW // 2)

        pltpu.sync_copy(idx_hbm.at[pl.ds(wid * _NCH, _NCH)], Iall)

        def start(c, Rw, gs):
            pltpu.async_copy(tab_hbm.at[Iall.at[c]], Rw, gs)

        start(0, R0, gs0)
        start(1, R1, gs1)

        def dst(c):
            return out_hbm.at[k0, pl.ds(rbase + c * _CH, _CH), pl.ds(0, C)]

        def step(c, Rw, gs, ds, Rn, gsn, dsn):
            pltpu.make_async_copy(tab_hbm.at[Iall.at[c]], Rw, gs).wait()
            pltpu.async_copy(Rw, dst(c), ds)

            @pl.when(c + 2 < NCHH)
            def _():
                @pl.when(c >= 2)
                def _():
                    pltpu.make_async_copy(Rn, dst(c), dsn).wait()
                start(c + 2, Rn, gsn)

        @pl.loop(0, NCHH, step=4)
        def lp(c):
            step(c + 0, R0, gs0, ds0, R2, gs2, ds2)
            step(c + 1, R1, gs1, ds1, R3, gs3, ds3)
            step(c + 2, R2, gs2, ds2, R0, gs0, ds0)
            step(c + 3, R3, gs3, ds3, R1, gs1, ds1)

        pltpu.make_async_copy(R0, dst(0), ds0).wait()
        pltpu.make_async_copy(R1, dst(0), ds1).wait()
        pltpu.make_async_copy(R2, dst(0), ds2).wait()
        pltpu.make_async_copy(R3, dst(0), ds3).wait()

    return pl.kernel(
        kbody,
        out_type=jax.ShapeDtypeStruct((_K, _NROW, 128), jnp.float32),
        mesh=mesh,
        compiler_params=pltpu.CompilerParams(use_tc_tiling_on_sc=False),
        scratch_types=(
            [pltpu.VMEM((NCHH, _CH), jnp.int32)]
            + [pltpu.VMEM((_CH, C), jnp.float32)] * 4
            + [pltpu.SemaphoreType.DMA] * 8
        ),
    )(tab, idxkm)


def _tc_conv_call(gath3, tab, W):
    O = W.shape[0]
    C = tab.shape[1]

    def body(g_ref, x_ref, w_ref, m_ref, p_ref):
        own = x_ref[...]
        if C < 128:
            own = jnp.concatenate(
                [own, jnp.zeros((_RN, 128 - C), jnp.float32)], axis=1)
        w = w_ref[...]
        lane = lax.broadcasted_iota(jnp.int32, (_K * _RN, 128), 1)
        d = (g_ref[...] - own[None, :, :]).reshape(_K * _RN, 128)
        d = jnp.where(lane < C, d, 0.0)
        p = _dot(d, w)
        m = p[0:_RN]
        for k in range(1, _K):
            m = jnp.maximum(m, p[k * _RN:(k + 1) * _RN])
        m_ref[...] = m
        p_ref[...] = jnp.stack([jnp.sum(p, 0), jnp.sum(p * p, 0)])[None]

    return pl.pallas_call(
        body,
        grid=(_NCB,),
        in_specs=[pl.BlockSpec((_K, _RN, 128), lambda i: (0, i, 0)),
                  pl.BlockSpec((_RN, C), lambda i: (i, 0)),
                  pl.BlockSpec((O, 128), lambda i: (0, 0))],
        out_specs=[pl.BlockSpec((_RN, O), lambda i: (i, 0)),
                   pl.BlockSpec((1, 2, O), lambda i: (i, 0, 0))],
        out_shape=[jax.ShapeDtypeStruct((_NROW, O), jnp.float32),
                   jax.ShapeDtypeStruct((_NCB, 2, O), jnp.float32)],
    )(gath3, tab, W)


def _tc_apply_call(M, P, g, b):
    O = M.shape[1]

    def body(m_ref, p_ref, g_ref, b_ref, o_ref):
        i = pl.program_id(0)
        scale, shift = _scale_shift(p_ref, g_ref, b_ref, _CNT2D)
        o_ref[...] = _row_mask(i, _R, _lrelu(m_ref[...] * scale[None, :]
                                             + shift[None, :]))

    return pl.pallas_call(
        body,
        grid=(_NBLK,),
        in_specs=[pl.BlockSpec((_R, O), lambda i: (i, 0)),
                  pl.BlockSpec((_NCB, 2, O), lambda i: (0, 0, 0)),
                  pl.BlockSpec((O,), lambda i: (0,)),
                  pl.BlockSpec((O,), lambda i: (0,))],
        out_specs=pl.BlockSpec((_R, O), lambda i: (i, 0)),
        out_shape=jax.ShapeDtypeStruct((_NROW, O), jnp.float32),
    )(M, P, g, b)


def _tc_apply4_call(M, P, g, b, x1, x2, x3, W5a, W5b, W5c, W5d):
    O = M.shape[1]
    On = W5a.shape[0]

    def body(m_ref, p_ref, g_ref, b_ref, x1_ref, x2_ref, x3_ref,
             w5a_ref, w5b_ref, w5c_ref, w5d_ref, y_ref, pout_ref):
        i = pl.program_id(0)
        scale, shift = _scale_shift(p_ref, g_ref, b_ref, _CNT2D)
        x4v = _row_mask(i, _R, _lrelu(m_ref[...] * scale[None, :]
                                      + shift[None, :]))
        y = (_dot(x1_ref[...], w5a_ref[...])
             + _dot(x2_ref[...], w5b_ref[...])
             + _dot(x3_ref[...], w5c_ref[...])
             + _dot(x4v, w5d_ref[...]))
        y_ref[...] = y
        pout_ref[...] = jnp.stack([jnp.sum(y, 0), jnp.sum(y * y, 0)])[None]

    return pl.pallas_call(
        body,
        grid=(_NBLK,),
        in_specs=[pl.BlockSpec((_R, O), lambda i: (i, 0)),
                  pl.BlockSpec((_NCB, 2, O), lambda i: (0, 0, 0)),
                  pl.BlockSpec((O,), lambda i: (0,)),
                  pl.BlockSpec((O,), lambda i: (0,)),
                  pl.BlockSpec((_R, 64), lambda i: (i, 0)),
                  pl.BlockSpec((_R, 64), lambda i: (i, 0)),
                  pl.BlockSpec((_R, 128), lambda i: (i, 0)),
                  pl.BlockSpec((On, 64), lambda i: (0, 0)),
                  pl.BlockSpec((On, 64), lambda i: (0, 0)),
                  pl.BlockSpec((On, 128), lambda i: (0, 0)),
                  pl.BlockSpec((On, 256), lambda i: (0, 0))],
        out_specs=[pl.BlockSpec((_R, On), lambda i: (i, 0)),
                   pl.BlockSpec((1, 2, On), lambda i: (i, 0, 0))],
        out_shape=[jax.ShapeDtypeStruct((_NROW, On), jnp.float32),
                   jax.ShapeDtypeStruct((_NBLK, 2, On), jnp.float32)],
    )(M, P, g, b, x1, x2, x3, W5a, W5b, W5c, W5d)


def _apply_mlp_call(Hpre, P, g, b, Wn):
    O = Hpre.shape[1]
    On = Wn.shape[0]
    npart = P.shape[0]

    def body(h_ref, p_ref, g_ref, b_ref, w_ref, y_ref, pout_ref):
        i = pl.program_id(0)
        scale, shift = _scale_shift(p_ref, g_ref, b_ref, _CNT1D)
        hv = _row_mask(i, _R, _lrelu(h_ref[...] * scale[None, :]
                                     + shift[None, :]))
        y = _dot(hv, w_ref[...])
        y_ref[...] = y
        pout_ref[...] = jnp.stack([jnp.sum(y, 0), jnp.sum(y * y, 0)])[None]

    return pl.pallas_call(
        body,
        grid=(_NBLK,),
        in_specs=[pl.BlockSpec((_R, O), lambda i: (i, 0)),
                  pl.BlockSpec((npart, 2, O), lambda i: (0, 0, 0)),
                  pl.BlockSpec((O,), lambda i: (0,)),
                  pl.BlockSpec((O,), lambda i: (0,)),
                  pl.BlockSpec((On, O), lambda i: (0, 0))],
        out_specs=[pl.BlockSpec((_R, On), lambda i: (i, 0)),
                   pl.BlockSpec((1, 2, On), lambda i: (i, 0, 0))],
        out_shape=[jax.ShapeDtypeStruct((_NROW, On), jnp.float32),
                   jax.ShapeDtypeStruct((_NBLK, 2, On), jnp.float32)],
    )(Hpre, P, g, b, Wn)


def _final_call(Hpre, P, g, b, W7):
    O = Hpre.shape[1]

    def body(h_ref, p_ref, g_ref, b_ref, w_ref, o_ref):
        scale, shift = _scale_shift(p_ref, g_ref, b_ref, _CNT1D)
        hv = _lrelu(h_ref[...] * scale[None, :] + shift[None, :])
        o_ref[...] = _lrelu(_dot(hv, w_ref[...]))

    return pl.pallas_call(
        body,
        grid=(_NBLK,),
        in_specs=[pl.BlockSpec((_R, O), lambda i: (i, 0)),
                  pl.BlockSpec((_NBLK, 2, O), lambda i: (0, 0, 0)),
                  pl.BlockSpec((O,), lambda i: (0,)),
                  pl.BlockSpec((O,), lambda i: (0,)),
                  pl.BlockSpec((1, O), lambda i: (0, 0))],
        out_specs=pl.BlockSpec((_R, 1), lambda i: (i, 0)),
        out_shape=jax.ShapeDtypeStruct((_NROW, 1), jnp.float32),
    )(Hpre, P, g, b, W7)


@jax.jit
def kernel(x, idx, W1, W2, W3, W4, W5, W6, W7,
           g1, b1, g2, b2, g3, b3, g4, b4, g5, b5, g6, b6):
    xT = jnp.transpose(x, (0, 2, 1))
    xT = jnp.pad(xT, ((0, 0), (0, _NPAD - _N), (0, 13)))
    T1 = xT.reshape(_NROW, 16)

    W1p = jnp.pad(W1, ((0, 0), (0, 125)))
    W2p = jnp.pad(W2, ((0, 0), (0, 64)))
    W3p = jnp.pad(W3, ((0, 0), (0, 64)))
    W5a = W5[:, 0:64]
    W5b = W5[:, 64:128]
    W5c = W5[:, 128:256]
    W5d = W5[:, 256:512]

    idxp = jnp.pad(idx, ((0, 0), (0, _NPAD - _N), (0, 0)), constant_values=_N)
    idxg = idxp + (jnp.arange(_B, dtype=jnp.int32) * _NPAD)[:, None, None]
    idxkm = jnp.transpose(idxg.reshape(_NROW, _K), (1, 0)).reshape(-1, _CH)

    def edge_layer(T, W, g, b):
        return _tc_conv_call(_sc_gather_rows(T, idxkm), T, W)

    M1, P1 = edge_layer(T1, W1p, g1, b1)
    T2 = _tc_apply_call(M1, P1, g1, b1)
    M2, P2 = edge_layer(T2, W2p, g2, b2)
    T3 = _tc_apply_call(M2, P2, g2, b2)
    M3, P3 = edge_layer(T3, W3p, g3, b3)
    T4 = _tc_apply_call(M3, P3, g3, b3)
    M4, P4 = edge_layer(T4, W4, g4, b4)
    h5pre, P5 = _tc_apply4_call(M4, P4, g4, b4, T2, T3, T4,
                                W5a, W5b, W5c, W5d)
    h6pre, P6 = _apply_mlp_call(h5pre, P5, g5, b5, W6)
    out = _final_call(h6pre, P6, g6, b6, W7)
    return out.reshape(_B, _NPAD, 1)[:, :_N, :]


# speedup vs baseline: 1.2688x; 1.0210x over previous
import jax
import jax.numpy as jnp
from jax import lax
from jax.experimental import pallas as pl
from jax.experimental.pallas import tpu as pltpu
from jax.experimental.pallas import tpu_sc as plsc

_B, _N, _K = 2, 10000, 16
_NPAD = 10240
_NROW = _B * _NPAD
_NE = _NROW * _K
_NW = 32
_EPW = _NE // _NW
_CH = 128
_NCH = _EPW // _CH
_RN = 256
_NCB = _NROW // _RN
_R = 1024
_NBLK = _NROW // _R
_EPS = 1e-5
_CNT2D = float(_B * _N * _K)
_CNT1D = float(_B * _N)


def _lrelu(v):
    return jnp.where(v >= 0, v, 0.2 * v)


def _row_mask(i, rows_per_block, v):
    rows = i * rows_per_block + lax.broadcasted_iota(
        jnp.int32, (rows_per_block, 1), 0)
    return jnp.where(rows % _NPAD < _N, v, 0.0)


def _dot(a, w):
    return lax.dot_general(a, w, (((1,), (1,)), ((), ())),
                           preferred_element_type=jnp.float32)


def _scale_shift(p_ref, g_ref, b_ref, cnt):
    p = p_ref[...]
    s = jnp.sum(p[:, 0, :], 0)
    s2 = jnp.sum(p[:, 1, :], 0)
    mean = s / cnt
    var = s2 / cnt - mean * mean
    scale = g_ref[...] * (1.0 / jnp.sqrt(var + _EPS))
    shift = b_ref[...] - mean * scale
    return scale, shift


def _sc_gather_rows(tab, idxkm):
    C = tab.shape[1]
    NBUF = 8 if C <= 64 else 5
    mesh = plsc.VectorSubcoreMesh(core_axis_name="c", subcore_axis_name="s")

    def body_with(tab_hbm, idx_hbm, out_hbm, Iall, Rws, gss, dss):
        cid = lax.axis_index("c")
        sid = lax.axis_index("s")
        wid = sid * 2 + cid
        k0 = wid // 2
        rbase = (wid % 2) * (_NROW // 2)

        pltpu.sync_copy(idx_hbm.at[pl.ds(wid * _NCH, _NCH)], Iall)

        def start(c, j):
            pltpu.async_copy(tab_hbm.at[Iall.at[c]], Rws[j], gss[j])

        for j in range(NBUF - 2):
            start(j, j)

        def dst(c):
            return out_hbm.at[k0, pl.ds(rbase + c * _CH, _CH), pl.ds(0, C)]

        def step(c, j):
            q = (j + NBUF - 2) % NBUF
            pltpu.make_async_copy(tab_hbm.at[Iall.at[c]], Rws[j],
                                  gss[j]).wait()
            pltpu.async_copy(Rws[j], dst(c), dss[j])

            @pl.when(c + NBUF - 2 < _NCH)
            def _():
                @pl.when(c >= 2)
                def _():
                    pltpu.make_async_copy(Rws[q], dst(c), dss[q]).wait()
                start(c + NBUF - 2, q)

        @pl.loop(0, _NCH, step=NBUF)
        def lp(c):
            for j in range(NBUF):
                step(c + j, j)

        for j in range(NBUF):
            pltpu.make_async_copy(Rws[j], dst(0), dss[j]).wait()

    if NBUF == 8:
        def kbody(tab_hbm, idx_hbm, out_hbm, Iall,
                  R0, R1, R2, R3, R4, R5, R6, R7,
                  g0, g1, g2, g3, g4, g5, g6, g7,
                  d0, d1, d2, d3, d4, d5, d6, d7):
            body_with(tab_hbm, idx_hbm, out_hbm, Iall,
                      [R0, R1, R2, R3, R4, R5, R6, R7],
                      [g0, g1, g2, g3, g4, g5, g6, g7],
                      [d0, d1, d2, d3, d4, d5, d6, d7])
    else:
        def kbody(tab_hbm, idx_hbm, out_hbm, Iall,
                  R0, R1, R2, R3, R4,
                  g0, g1, g2, g3, g4,
                  d0, d1, d2, d3, d4):
            body_with(tab_hbm, idx_hbm, out_hbm, Iall,
                      [R0, R1, R2, R3, R4],
                      [g0, g1, g2, g3, g4],
                      [d0, d1, d2, d3, d4])

    return pl.kernel(
        kbody,
        out_type=jax.ShapeDtypeStruct((_K, _NROW, 128), jnp.float32),
        mesh=mesh,
        compiler_params=pltpu.CompilerParams(use_tc_tiling_on_sc=False),
        scratch_types=(
            [pltpu.VMEM((_NCH, _CH), jnp.int32)]
            + [pltpu.VMEM((_CH, C), jnp.float32)] * NBUF
            + [pltpu.SemaphoreType.DMA] * (2 * NBUF)
        ),
    )(tab, idxkm)


def _tc_conv_call(gath3, tab, W):
    O = W.shape[0]
    C = tab.shape[1]

    def body(g_ref, x_ref, w_ref, m_ref, p_ref):
        own = x_ref[...]
        if C < 128:
            own = jnp.concatenate(
                [own, jnp.zeros((_RN, 128 - C), jnp.float32)], axis=1)
        w = w_ref[...]
        lane = lax.broadcasted_iota(jnp.int32, (_K * _RN, 128), 1)
        d = (g_ref[...] - own[None, :, :]).reshape(_K * _RN, 128)
        d = jnp.where(lane < C, d, 0.0)
        p = _dot(d, w)
        m = p[0:_RN]
        for k in range(1, _K):
            m = jnp.maximum(m, p[k * _RN:(k + 1) * _RN])
        m_ref[...] = m
        p_ref[...] = jnp.stack([jnp.sum(p, 0), jnp.sum(p * p, 0)])[None]

    return pl.pallas_call(
        body,
        grid=(_NCB,),
        in_specs=[pl.BlockSpec((_K, _RN, 128), lambda i: (0, i, 0)),
                  pl.BlockSpec((_RN, C), lambda i: (i, 0)),
                  pl.BlockSpec((O, 128), lambda i: (0, 0))],
        out_specs=[pl.BlockSpec((_RN, O), lambda i: (i, 0)),
                   pl.BlockSpec((1, 2, O), lambda i: (i, 0, 0))],
        out_shape=[jax.ShapeDtypeStruct((_NROW, O), jnp.float32),
                   jax.ShapeDtypeStruct((_NCB, 2, O), jnp.float32)],
    )(gath3, tab, W)


def _tc_apply_call(M, P, g, b):
    O = M.shape[1]

    def body(m_ref, p_ref, g_ref, b_ref, o_ref):
        i = pl.program_id(0)
        scale, shift = _scale_shift(p_ref, g_ref, b_ref, _CNT2D)
        o_ref[...] = _row_mask(i, _R, _lrelu(m_ref[...] * scale[None, :]
                                             + shift[None, :]))

    return pl.pallas_call(
        body,
        grid=(_NBLK,),
        in_specs=[pl.BlockSpec((_R, O), lambda i: (i, 0)),
                  pl.BlockSpec((_NCB, 2, O), lambda i: (0, 0, 0)),
                  pl.BlockSpec((O,), lambda i: (0,)),
                  pl.BlockSpec((O,), lambda i: (0,))],
        out_specs=pl.BlockSpec((_R, O), lambda i: (i, 0)),
        out_shape=jax.ShapeDtypeStruct((_NROW, O), jnp.float32),
    )(M, P, g, b)


def _tc_apply4_call(M, P, g, b, x1, x2, x3, W5a, W5b, W5c, W5d):
    O = M.shape[1]
    On = W5a.shape[0]

    def body(m_ref, p_ref, g_ref, b_ref, x1_ref, x2_ref, x3_ref,
             w5a_ref, w5b_ref, w5c_ref, w5d_ref, y_ref, pout_ref):
        i = pl.program_id(0)
        scale, shift = _scale_shift(p_ref, g_ref, b_ref, _CNT2D)
        x4v = _row_mask(i, _R, _lrelu(m_ref[...] * scale[None, :]
                                      + shift[None, :]))
        y = (_dot(x1_ref[...], w5a_ref[...])
             + _dot(x2_ref[...], w5b_ref[...])
             + _dot(x3_ref[...], w5c_ref[...])
             + _dot(x4v, w5d_ref[...]))
        y_ref[...] = y
        pout_ref[...] = jnp.stack([jnp.sum(y, 0), jnp.sum(y * y, 0)])[None]

    return pl.pallas_call(
        body,
        grid=(_NBLK,),
        in_specs=[pl.BlockSpec((_R, O), lambda i: (i, 0)),
                  pl.BlockSpec((_NCB, 2, O), lambda i: (0, 0, 0)),
                  pl.BlockSpec((O,), lambda i: (0,)),
                  pl.BlockSpec((O,), lambda i: (0,)),
                  pl.BlockSpec((_R, 64), lambda i: (i, 0)),
                  pl.BlockSpec((_R, 64), lambda i: (i, 0)),
                  pl.BlockSpec((_R, 128), lambda i: (i, 0)),
                  pl.BlockSpec((On, 64), lambda i: (0, 0)),
                  pl.BlockSpec((On, 64), lambda i: (0, 0)),
                  pl.BlockSpec((On, 128), lambda i: (0, 0)),
                  pl.BlockSpec((On, 256), lambda i: (0, 0))],
        out_specs=[pl.BlockSpec((_R, On), lambda i: (i, 0)),
                   pl.BlockSpec((1, 2, On), lambda i: (i, 0, 0))],
        out_shape=[jax.ShapeDtypeStruct((_NROW, On), jnp.float32),
                   jax.ShapeDtypeStruct((_NBLK, 2, On), jnp.float32)],
    )(M, P, g, b, x1, x2, x3, W5a, W5b, W5c, W5d)


def _apply_mlp_call(Hpre, P, g, b, Wn):
    O = Hpre.shape[1]
    On = Wn.shape[0]
    npart = P.shape[0]

    def body(h_ref, p_ref, g_ref, b_ref, w_ref, y_ref, pout_ref):
        i = pl.program_id(0)
        scale, shift = _scale_shift(p_ref, g_ref, b_ref, _CNT1D)
        hv = _row_mask(i, _R, _lrelu(h_ref[...] * scale[None, :]
                                     + shift[None, :]))
        y = _dot(hv, w_ref[...])
        y_ref[...] = y
        pout_ref[...] = jnp.stack([jnp.sum(y, 0), jnp.sum(y * y, 0)])[None]

    return pl.pallas_call(
        body,
        grid=(_NBLK,),
        in_specs=[pl.BlockSpec((_R, O), lambda i: (i, 0)),
                  pl.BlockSpec((npart, 2, O), lambda i: (0, 0, 0)),
                  pl.BlockSpec((O,), lambda i: (0,)),
                  pl.BlockSpec((O,), lambda i: (0,)),
                  pl.BlockSpec((On, O), lambda i: (0, 0))],
        out_specs=[pl.BlockSpec((_R, On), lambda i: (i, 0)),
                   pl.BlockSpec((1, 2, On), lambda i: (i, 0, 0))],
        out_shape=[jax.ShapeDtypeStruct((_NROW, On), jnp.float32),
                   jax.ShapeDtypeStruct((_NBLK, 2, On), jnp.float32)],
    )(Hpre, P, g, b, Wn)


def _final_call(Hpre, P, g, b, W7):
    O = Hpre.shape[1]

    def body(h_ref, p_ref, g_ref, b_ref, w_ref, o_ref):
        scale, shift = _scale_shift(p_ref, g_ref, b_ref, _CNT1D)
        hv = _lrelu(h_ref[...] * scale[None, :] + shift[None, :])
        o_ref[...] = _lrelu(_dot(hv, w_ref[...]))

    return pl.pallas_call(
        body,
        grid=(_NBLK,),
        in_specs=[pl.BlockSpec((_R, O), lambda i: (i, 0)),
                  pl.BlockSpec((_NBLK, 2, O), lambda i: (0, 0, 0)),
                  pl.BlockSpec((O,), lambda i: (0,)),
                  pl.BlockSpec((O,), lambda i: (0,)),
                  pl.BlockSpec((1, O), lambda i: (0, 0))],
        out_specs=pl.BlockSpec((_R, 1), lambda i: (i, 0)),
        out_shape=jax.ShapeDtypeStruct((_NROW, 1), jnp.float32),
    )(Hpre, P, g, b, W7)


@jax.jit
def kernel(x, idx, W1, W2, W3, W4, W5, W6, W7,
           g1, b1, g2, b2, g3, b3, g4, b4, g5, b5, g6, b6):
    xT = jnp.transpose(x, (0, 2, 1))
    xT = jnp.pad(xT, ((0, 0), (0, _NPAD - _N), (0, 13)))
    T1 = xT.reshape(_NROW, 16)

    W1p = jnp.pad(W1, ((0, 0), (0, 125)))
    W2p = jnp.pad(W2, ((0, 0), (0, 64)))
    W3p = jnp.pad(W3, ((0, 0), (0, 64)))
    W5a = W5[:, 0:64]
    W5b = W5[:, 64:128]
    W5c = W5[:, 128:256]
    W5d = W5[:, 256:512]

    idxp = jnp.pad(idx, ((0, 0), (0, _NPAD - _N), (0, 0)), constant_values=_N)
    idxg = idxp + (jnp.arange(_B, dtype=jnp.int32) * _NPAD)[:, None, None]
    idxkm = jnp.transpose(idxg.reshape(_NROW, _K), (1, 0)).reshape(-1, _CH)

    def edge_layer(T, W, g, b):
        return _tc_conv_call(_sc_gather_rows(T, idxkm), T, W)

    M1, P1 = edge_layer(T1, W1p, g1, b1)
    T2 = _tc_apply_call(M1, P1, g1, b1)
    M2, P2 = edge_layer(T2, W2p, g2, b2)
    T3 = _tc_apply_call(M2, P2, g2, b2)
    M3, P3 = edge_layer(T3, W3p, g3, b3)
    T4 = _tc_apply_call(M3, P3, g3, b3)
    M4, P4 = edge_layer(T4, W4, g4, b4)
    h5pre, P5 = _tc_apply4_call(M4, P4, g4, b4, T2, T3, T4,
                                W5a, W5b, W5c, W5d)
    h6pre, P6 = _apply_mlp_call(h5pre, P5, g5, b5, W6)
    out = _final_call(h6pre, P6, g6, b6, W7)
    return out.reshape(_B, _NPAD, 1)[:, :_N, :]
